# Initial kernel scaffold; baseline (speedup 1.0000x reference)
import jax
import jax.numpy as jnp
from jax.experimental import pallas as pl


def kernel(x, a, edge_attr, u, W, b, edge_index, batch):
    raise NotImplementedError("write your pallas kernel here")


# speedup vs baseline: 5.2902x; 5.2902x over previous
import functools

import jax
import jax.numpy as jnp
from jax import lax
from jax.experimental import pallas as pl
from jax.experimental.pallas import tpu as pltpu
from jax.experimental.pallas import tpu_sc as plsc

N = 10000
E = 320000
FX = 128
FE = 16
FU = 64
B = 64
FOUT = 128
FIN = FE + 2 * FX + FU

NC = 2
NS = 16
NW = NC * NS
EPW = E // NW
K = 80
STEPS = EPW // K
CH = 400
SPC = CH // K
NCH = EPW // CH
N_ACC = 10240
ROWS_PER_TILE = N_ACC // NS
CW = 8

def _sc_scatter_body(dest_hbm, edge_hbm, zeros_hbm, zeros_cnt_hbm, ones_hbm,
                     sums_out, cnt_out,
                     idx_v, rows_v, ones_v, sums_acc, cnt_acc, sem):
    cid = lax.axis_index("c")
    sid = lax.axis_index("s")
    wid = cid * NS + sid

    row0 = sid * ROWS_PER_TILE
    pltpu.sync_copy(zeros_hbm, sums_acc.at[pl.ds(row0, ROWS_PER_TILE)])
    pltpu.sync_copy(zeros_cnt_hbm, cnt_acc.at[pl.ds(row0, ROWS_PER_TILE)])
    pltpu.sync_copy(ones_hbm, ones_v)
    pltpu.sync_copy(dest_hbm.at[wid], idx_v)
    plsc.subcore_barrier()

    base = wid * EPW

    def chunk_body(c, carry):
        buf = lax.rem(c, 2)
        pltpu.sync_copy(edge_hbm.at[pl.ds(base + c * CH, CH)], rows_v.at[buf])
        for j in range(SPC):
            step = c * SPC + j
            idx_row = idx_v.at[step]
            pltpu.sync_copy(rows_v.at[buf, pl.ds(j * K, K)],
                            sums_acc.at[idx_row], add=True)
            pltpu.sync_copy(ones_v, cnt_acc.at[idx_row], add=True)
        return carry

    lax.fori_loop(0, NCH, chunk_body, 0)
    plsc.subcore_barrier()

    pltpu.sync_copy(sums_acc.at[pl.ds(row0, ROWS_PER_TILE)],
                    sums_out.at[cid, pl.ds(row0, ROWS_PER_TILE)])
    pltpu.sync_copy(cnt_acc.at[pl.ds(row0, ROWS_PER_TILE)],
                    cnt_out.at[cid, pl.ds(row0, ROWS_PER_TILE)])


@functools.lru_cache(maxsize=1)
def _sc_scatter_call():
    mesh = plsc.VectorSubcoreMesh(core_axis_name="c", subcore_axis_name="s")
    return pl.kernel(
        _sc_scatter_body,
        out_type=(
            jax.ShapeDtypeStruct((NC, N_ACC, FE), jnp.float32),
            jax.ShapeDtypeStruct((NC, N_ACC, CW), jnp.float32),
        ),
        mesh=mesh,
        scratch_types=(
            pltpu.VMEM((STEPS, K), jnp.int32),
            pltpu.VMEM((2, CH, FE), jnp.float32),
            pltpu.VMEM((K, CW), jnp.float32),
            pltpu.VMEM_SHARED((N_ACC, FE), jnp.float32),
            pltpu.VMEM_SHARED((N_ACC, CW), jnp.float32),
            pltpu.SemaphoreType.DMA,
        ),
        compiler_params=pltpu.CompilerParams(use_tc_tiling_on_sc=False),
    )


BR = 1000


def _tc_body(x_ref, a_ref, s_ref, c_ref, bt_ref, u_ref, w_ref, b_ref, o_ref):
    w = w_ref[:]
    hp = jax.lax.Precision.HIGHEST
    acc = jnp.dot(x_ref[:], w[0:FX], precision=hp,
                  preferred_element_type=jnp.float32)
    acc += jnp.dot(a_ref[:], w[FX:2 * FX], precision=hp,
                   preferred_element_type=jnp.float32)
    sums = s_ref[0] + s_ref[1]
    cnt = c_ref[0, :, 0:1] + c_ref[1, :, 0:1]
    e_agg = sums / jnp.maximum(cnt, 1.0)
    acc += jnp.dot(e_agg, w[2 * FX:2 * FX + FE], precision=hp,
                   preferred_element_type=jnp.float32)
    onehot = (bt_ref[:] == lax.broadcasted_iota(jnp.int32, (BR, B), 1)
              ).astype(jnp.float32)
    uw = jnp.dot(u_ref[:], w[2 * FX + FE:FIN], precision=hp,
                 preferred_element_type=jnp.float32)
    acc += jnp.dot(onehot, uw, precision=hp,
                   preferred_element_type=jnp.float32)
    o_ref[:] = jnp.maximum(acc + b_ref[:], 0.0)


_tc_call = pl.pallas_call(
    _tc_body,
    grid=(N // BR,),
    in_specs=[
        pl.BlockSpec((BR, FX), lambda i: (i, 0)),
        pl.BlockSpec((BR, FX), lambda i: (i, 0)),
        pl.BlockSpec((NC, BR, FE), lambda i: (0, i, 0)),
        pl.BlockSpec((NC, BR, CW), lambda i: (0, i, 0)),
        pl.BlockSpec((BR, 1), lambda i: (i, 0)),
        pl.BlockSpec((B, FU), lambda i: (0, 0)),
        pl.BlockSpec((FIN, FOUT), lambda i: (0, 0)),
        pl.BlockSpec((1, FOUT), lambda i: (0, 0)),
    ],
    out_specs=pl.BlockSpec((BR, FOUT), lambda i: (i, 0)),
    out_shape=jax.ShapeDtypeStruct((N, FOUT), jnp.float32),
)


def kernel(x, a, edge_attr, u, W, b, edge_index, batch):
    dest3 = edge_index[1].reshape(NW, STEPS, K)
    zeros_init = jnp.zeros((ROWS_PER_TILE, FE), jnp.float32)
    zeros_cnt = jnp.zeros((ROWS_PER_TILE, CW), jnp.float32)
    ones_src = jnp.ones((K, CW), jnp.float32)
    sums2, cnt2 = _sc_scatter_call()(dest3, edge_attr, zeros_init, zeros_cnt,
                                     ones_src)
    return _tc_call(x, a, sums2, cnt2, batch.reshape(N, 1), u, W,
                    b.reshape(1, FOUT))


# speedup vs baseline: 6.0271x; 1.1393x over previous
import functools

import jax
import jax.numpy as jnp
from jax import lax
from jax.experimental import pallas as pl
from jax.experimental.pallas import tpu as pltpu
from jax.experimental.pallas import tpu_sc as plsc

N = 10000
E = 320000
FX = 128
FE = 16
FU = 64
B = 64
FOUT = 128
FIN = FE + 2 * FX + FU

NC = 2
NS = 16
NW = NC * NS
EPW = E // NW
CH = 2000
NCH = EPW // CH
N_ACC = 10240
ROWS_PER_TILE = N_ACC // NS
CW = 8

def _sc_scatter_body(ei_hbm, edge_hbm, zeros_hbm, zeros_cnt_hbm, ones_hbm,
                     sums_out, cnt_out,
                     idx_v, rows_v, ones_v, sums_acc, cnt_acc,
                     sem_a, sem_b, sem_i):
    cid = lax.axis_index("c")
    sid = lax.axis_index("s")
    wid = cid * NS + sid

    row0 = sid * ROWS_PER_TILE
    pltpu.sync_copy(zeros_hbm, sums_acc.at[pl.ds(row0, ROWS_PER_TILE)])
    pltpu.sync_copy(zeros_cnt_hbm, cnt_acc.at[pl.ds(row0, ROWS_PER_TILE)])
    base = wid * EPW
    idx_cp = pltpu.async_copy(ei_hbm.at[1, pl.ds(base, EPW)], idx_v, sem_i)
    pltpu.sync_copy(ones_hbm, ones_v)
    idx_cp.wait()
    plsc.subcore_barrier()

    sems = (sem_a, sem_b)
    cps = [None, None]
    cps[0] = pltpu.async_copy(edge_hbm.at[pl.ds(base, CH)], rows_v.at[0],
                              sems[0])
    for c in range(NCH):
        buf = c % 2
        if c + 1 < NCH:
            nxt = (c + 1) % 2
            cps[nxt] = pltpu.async_copy(
                edge_hbm.at[pl.ds(base + (c + 1) * CH, CH)], rows_v.at[nxt],
                sems[nxt])
        cps[buf].wait()
        idx_row = idx_v.at[pl.ds(c * CH, CH)]
        pltpu.sync_copy(rows_v.at[buf], sums_acc.at[idx_row], add=True)
        pltpu.sync_copy(ones_v, cnt_acc.at[idx_row], add=True)
    plsc.subcore_barrier()

    pltpu.sync_copy(sums_acc.at[pl.ds(row0, ROWS_PER_TILE)],
                    sums_out.at[cid, pl.ds(row0, ROWS_PER_TILE)])
    pltpu.sync_copy(cnt_acc.at[pl.ds(row0, ROWS_PER_TILE)],
                    cnt_out.at[cid, pl.ds(row0, ROWS_PER_TILE)])


@functools.lru_cache(maxsize=1)
def _sc_scatter_call():
    mesh = plsc.VectorSubcoreMesh(core_axis_name="c", subcore_axis_name="s")
    return pl.kernel(
        _sc_scatter_body,
        out_type=(
            jax.ShapeDtypeStruct((NC, N_ACC, FE), jnp.float32),
            jax.ShapeDtypeStruct((NC, N_ACC, CW), jnp.float32),
        ),
        mesh=mesh,
        scratch_types=(
            pltpu.VMEM((EPW,), jnp.int32),
            pltpu.VMEM((2, CH, FE), jnp.float32),
            pltpu.VMEM((CH, CW), jnp.float32),
            pltpu.VMEM_SHARED((N_ACC, FE), jnp.float32),
            pltpu.VMEM_SHARED((N_ACC, CW), jnp.float32),
            pltpu.SemaphoreType.DMA,
            pltpu.SemaphoreType.DMA,
            pltpu.SemaphoreType.DMA,
        ),
        compiler_params=pltpu.CompilerParams(use_tc_tiling_on_sc=False),
    )


BR = 1000


def _tc_body(x_ref, a_ref, s_ref, c_ref, bt_ref, u_ref, w_ref, b_ref, o_ref):
    w = w_ref[:]
    hp = jax.lax.Precision.HIGHEST
    acc = jnp.dot(x_ref[:], w[0:FX], precision=hp,
                  preferred_element_type=jnp.float32)
    acc += jnp.dot(a_ref[:], w[FX:2 * FX], precision=hp,
                   preferred_element_type=jnp.float32)
    sums = s_ref[0] + s_ref[1]
    cnt = c_ref[0, :, 0:1] + c_ref[1, :, 0:1]
    e_agg = sums / jnp.maximum(cnt, 1.0)
    acc += jnp.dot(e_agg, w[2 * FX:2 * FX + FE], precision=hp,
                   preferred_element_type=jnp.float32)
    onehot = (bt_ref[:] == lax.broadcasted_iota(jnp.int32, (BR, B), 1)
              ).astype(jnp.float32)
    uw = jnp.dot(u_ref[:], w[2 * FX + FE:FIN], precision=hp,
                 preferred_element_type=jnp.float32)
    acc += jnp.dot(onehot, uw, precision=hp,
                   preferred_element_type=jnp.float32)
    o_ref[:] = jnp.maximum(acc + b_ref[:], 0.0)


_tc_call = pl.pallas_call(
    _tc_body,
    grid=(N // BR,),
    in_specs=[
        pl.BlockSpec((BR, FX), lambda i: (i, 0)),
        pl.BlockSpec((BR, FX), lambda i: (i, 0)),
        pl.BlockSpec((NC, BR, FE), lambda i: (0, i, 0)),
        pl.BlockSpec((NC, BR, CW), lambda i: (0, i, 0)),
        pl.BlockSpec((BR, 1), lambda i: (i, 0)),
        pl.BlockSpec((B, FU), lambda i: (0, 0)),
        pl.BlockSpec((FIN, FOUT), lambda i: (0, 0)),
        pl.BlockSpec((1, FOUT), lambda i: (0, 0)),
    ],
    out_specs=pl.BlockSpec((BR, FOUT), lambda i: (i, 0)),
    out_shape=jax.ShapeDtypeStruct((N, FOUT), jnp.float32),
)


def kernel(x, a, edge_attr, u, W, b, edge_index, batch):
    zeros_init = jnp.zeros((ROWS_PER_TILE, FE), jnp.float32)
    zeros_cnt = jnp.zeros((ROWS_PER_TILE, CW), jnp.float32)
    ones_src = jnp.ones((CH, CW), jnp.float32)
    sums2, cnt2 = _sc_scatter_call()(edge_index, edge_attr, zeros_init,
                                     zeros_cnt, ones_src)
    return _tc_call(x, a, sums2, cnt2, batch.reshape(N, 1), u, W,
                    b.reshape(1, FOUT))


# speedup vs baseline: 6.7497x; 1.1199x over previous
import functools

import jax
import jax.numpy as jnp
from jax import lax
from jax.experimental import pallas as pl
from jax.experimental.pallas import tpu as pltpu
from jax.experimental.pallas import tpu_sc as plsc

N = 10000
E = 320000
FX = 128
FE = 16
FU = 64
B = 64
FOUT = 128
FIN = FE + 2 * FX + FU

NC = 2
NS = 16
NW = NC * NS
EPW = E // NW
CH = 2000
NCH = EPW // CH
N_ACC = 10240
ROWS_PER_TILE = N_ACC // NS
CW = 8

def _sc_scatter_body(ei_hbm, edge_hbm, zeros_hbm, zeros_cnt_hbm, ones_hbm,
                     sums_out, cnt_out,
                     idx_v, rows_v, ones_v, sums_acc, cnt_acc,
                     sem_a, sem_b, sem_i):
    cid = lax.axis_index("c")
    sid = lax.axis_index("s")
    wid = cid * NS + sid

    row0 = sid * ROWS_PER_TILE
    pltpu.sync_copy(zeros_hbm, sums_acc.at[pl.ds(row0, ROWS_PER_TILE)])
    pltpu.sync_copy(zeros_cnt_hbm, cnt_acc.at[pl.ds(row0, ROWS_PER_TILE)])
    base = wid * EPW
    idx_cp = pltpu.async_copy(ei_hbm.at[1, pl.ds(base, EPW)], idx_v, sem_i)
    pltpu.sync_copy(ones_hbm, ones_v)
    idx_cp.wait()
    plsc.subcore_barrier()

    sems = (sem_a, sem_b)
    cps = [None, None]
    cps[0] = pltpu.async_copy(edge_hbm.at[pl.ds(base, CH)], rows_v.at[0],
                              sems[0])
    for c in range(NCH):
        buf = c % 2
        if c + 1 < NCH:
            nxt = (c + 1) % 2
            cps[nxt] = pltpu.async_copy(
                edge_hbm.at[pl.ds(base + (c + 1) * CH, CH)], rows_v.at[nxt],
                sems[nxt])
        cps[buf].wait()
        idx_row = idx_v.at[pl.ds(c * CH, CH)]
        pltpu.sync_copy(rows_v.at[buf], sums_acc.at[idx_row], add=True)
        pltpu.sync_copy(ones_v, cnt_acc.at[idx_row], add=True)
    plsc.subcore_barrier()

    pltpu.sync_copy(sums_acc.at[pl.ds(row0, ROWS_PER_TILE)],
                    sums_out.at[cid, pl.ds(row0, ROWS_PER_TILE)])
    pltpu.sync_copy(cnt_acc.at[pl.ds(row0, ROWS_PER_TILE)],
                    cnt_out.at[cid, pl.ds(row0, ROWS_PER_TILE)])


@functools.lru_cache(maxsize=1)
def _sc_scatter_call():
    mesh = plsc.VectorSubcoreMesh(core_axis_name="c", subcore_axis_name="s")
    return pl.kernel(
        _sc_scatter_body,
        out_type=(
            jax.ShapeDtypeStruct((NC, N_ACC, FE), jnp.float32),
            jax.ShapeDtypeStruct((NC, N_ACC, CW), jnp.float32),
        ),
        mesh=mesh,
        scratch_types=(
            pltpu.VMEM((EPW,), jnp.int32),
            pltpu.VMEM((2, CH, FE), jnp.float32),
            pltpu.VMEM((CH, CW), jnp.float32),
            pltpu.VMEM_SHARED((N_ACC, FE), jnp.float32),
            pltpu.VMEM_SHARED((N_ACC, CW), jnp.float32),
            pltpu.SemaphoreType.DMA,
            pltpu.SemaphoreType.DMA,
            pltpu.SemaphoreType.DMA,
        ),
        compiler_params=pltpu.CompilerParams(use_tc_tiling_on_sc=False),
    )


BR = 1000


def _tc_partial_body(x_ref, a_ref, bt_ref, u_ref, w_ref, b_ref, o_ref):
    w = w_ref[:]
    hp = jax.lax.Precision.DEFAULT
    acc = jnp.dot(x_ref[:], w[0:FX], precision=hp,
                  preferred_element_type=jnp.float32)
    acc += jnp.dot(a_ref[:], w[FX:2 * FX], precision=hp,
                   preferred_element_type=jnp.float32)
    onehot = (bt_ref[:] == lax.broadcasted_iota(jnp.int32, (BR, B), 1)
              ).astype(jnp.float32)
    uw = jnp.dot(u_ref[:], w[2 * FX + FE:FIN], precision=hp,
                 preferred_element_type=jnp.float32)
    acc += jnp.dot(onehot, uw, precision=hp,
                   preferred_element_type=jnp.float32)
    o_ref[:] = acc + b_ref[:]


_tc_partial = pl.pallas_call(
    _tc_partial_body,
    grid=(N // BR,),
    in_specs=[
        pl.BlockSpec((BR, FX), lambda i: (i, 0)),
        pl.BlockSpec((BR, FX), lambda i: (i, 0)),
        pl.BlockSpec((BR, 1), lambda i: (i, 0)),
        pl.BlockSpec((B, FU), lambda i: (0, 0)),
        pl.BlockSpec((FIN, FOUT), lambda i: (0, 0)),
        pl.BlockSpec((1, FOUT), lambda i: (0, 0)),
    ],
    out_specs=pl.BlockSpec((BR, FOUT), lambda i: (i, 0)),
    out_shape=jax.ShapeDtypeStruct((N, FOUT), jnp.float32),
)


def _tc_final_body(p_ref, s_ref, c_ref, w_ref, o_ref):
    hp = jax.lax.Precision.DEFAULT
    sums = s_ref[0] + s_ref[1]
    cnt = c_ref[0, :, 0:1] + c_ref[1, :, 0:1]
    e_agg = sums / jnp.maximum(cnt, 1.0)
    acc = p_ref[:] + jnp.dot(e_agg, w_ref[:], precision=hp,
                             preferred_element_type=jnp.float32)
    o_ref[:] = jnp.maximum(acc, 0.0)


_tc_final = pl.pallas_call(
    _tc_final_body,
    grid=(N // BR,),
    in_specs=[
        pl.BlockSpec((BR, FOUT), lambda i: (i, 0)),
        pl.BlockSpec((NC, BR, FE), lambda i: (0, i, 0)),
        pl.BlockSpec((NC, BR, CW), lambda i: (0, i, 0)),
        pl.BlockSpec((FE, FOUT), lambda i: (0, 0)),
    ],
    out_specs=pl.BlockSpec((BR, FOUT), lambda i: (i, 0)),
    out_shape=jax.ShapeDtypeStruct((N, FOUT), jnp.float32),
)


def kernel(x, a, edge_attr, u, W, b, edge_index, batch):
    zeros_init = jnp.zeros((ROWS_PER_TILE, FE), jnp.float32)
    zeros_cnt = jnp.zeros((ROWS_PER_TILE, CW), jnp.float32)
    ones_src = jnp.ones((CH, CW), jnp.float32)
    sums2, cnt2 = _sc_scatter_call()(edge_index, edge_attr, zeros_init,
                                     zeros_cnt, ones_src)
    p = _tc_partial(x, a, batch.reshape(N, 1), u, W, b.reshape(1, FOUT))
    return _tc_final(p, sums2, cnt2, W[2 * FX:2 * FX + FE])
